# baseline (speedup 1.0000x reference)
import jax
import jax.numpy as jnp
import numpy as np
from jax import lax
from jax.experimental import pallas as pl

B, C, H, W = 8, 128, 64, 64
D, K = 64, 512
HW = H * W
P = 4096
TB = HW // P
DA = D + 8


def _dot(a, b):
    return lax.dot_general(a, b, (((1,), (0,)), ((), ())),
                           preferred_element_type=jnp.float32)


def _vq_body(x_ref, w_ref, b_ref, et2_ref, ea_ref, q_ref, ind_ref, diff_ref):
    bix = pl.program_id(0)
    t = pl.program_id(1)
    x_blk = x_ref[0].reshape(C, P)
    xf = _dot(w_ref[...], x_blk) + b_ref[...]
    f2 = jnp.sum(xf * xf, axis=0, keepdims=True)
    et2 = et2_ref[...]
    e2c = jnp.sum(et2 * et2, axis=1, keepdims=True) * 0.25
    mmneg = _dot(et2, xf)
    dist = (f2 + mmneg) + e2c
    dmin = jnp.min(dist, axis=0, keepdims=True)
    eq_f = (dist == dmin).astype(jnp.float32)
    out = _dot(ea_ref[...], eq_f)
    q_ref[0] = out[0:D].reshape(D, H, W)
    tail = out[D:DA]
    ind_ref[0] = (2.0 * tail[0:1] + tail[1:2]).astype(jnp.int32)

    @pl.when((bix == 0) & (t == 0))
    def _init():
        diff_ref[...] = jnp.zeros_like(diff_ref)

    diff_ref[...] += jnp.sum(dmin).reshape(1, 1)


def kernel(x, conv_w, conv_b, embed):
    et2 = (-2.0 * embed).T
    kk = np.arange(K, dtype=np.float32)
    e_aug = jnp.concatenate([
        embed,
        jnp.asarray(np.floor(kk / 2.0))[None, :],
        jnp.asarray(kk % 2.0)[None, :],
        jnp.zeros((DA - D - 2, K), jnp.float32),
    ], axis=0)
    q, ind, diff = pl.pallas_call(
        _vq_body,
        grid=(B, TB),
        in_specs=[
            pl.BlockSpec((1, C, H, W), lambda b, t: (b, 0, 0, 0)),
            pl.BlockSpec((D, C), lambda b, t: (0, 0)),
            pl.BlockSpec((D, 1), lambda b, t: (0, 0)),
            pl.BlockSpec((K, D), lambda b, t: (0, 0)),
            pl.BlockSpec((DA, K), lambda b, t: (0, 0)),
        ],
        out_specs=[
            pl.BlockSpec((1, D, H, W), lambda b, t: (b, 0, 0, 0)),
            pl.BlockSpec((1, 1, P), lambda b, t: (b, 0, t)),
            pl.BlockSpec((1, 1), lambda b, t: (0, 0)),
        ],
        out_shape=[
            jax.ShapeDtypeStruct((B, D, H, W), jnp.float32),
            jax.ShapeDtypeStruct((B, 1, HW), jnp.int32),
            jax.ShapeDtypeStruct((1, 1), jnp.float32),
        ],
    )(x, conv_w, conv_b.reshape(D, 1), et2, e_aug)
    quantize = q
    embed_ind = ind.reshape(B, H, W)
    diff_s = diff[0, 0] / (B * HW * D)
    return quantize, diff_s, embed_ind


# speedup vs baseline: 1.4125x; 1.4125x over previous
import jax
import jax.numpy as jnp
import numpy as np
from jax import lax
from jax.experimental import pallas as pl

B, C, H, W = 8, 128, 64, 64
D, K = 64, 512
HW = H * W
P = 4096
TB = HW // P
DA = D + 8


def _dot(a, b):
    return lax.dot_general(a, b, (((1,), (0,)), ((), ())),
                           preferred_element_type=jnp.float32)


def _vq_body(x_ref, w_ref, b_ref, et2_ref, ea_ref, q_ref, ind_ref, diff_ref):
    bix = pl.program_id(0)
    t = pl.program_id(1)
    x_blk = x_ref[0]
    xf = _dot(w_ref[...], x_blk) + b_ref[...]
    f2 = jnp.sum(xf * xf, axis=0, keepdims=True)
    et2 = et2_ref[...]
    e2c = jnp.sum(et2 * et2, axis=1, keepdims=True) * 0.25
    mmneg = _dot(et2, xf)
    dist = (f2 + mmneg) + e2c
    dmin = jnp.min(dist, axis=0, keepdims=True)
    eq_f = (dist == dmin).astype(jnp.float32)
    out = _dot(ea_ref[...], eq_f)
    q_ref[0] = out[0:D].reshape(D, H, W)
    tail = out[D:DA]
    ind_ref[0] = (2.0 * tail[0:1] + tail[1:2]).astype(jnp.int32)

    @pl.when((bix == 0) & (t == 0))
    def _init():
        diff_ref[...] = jnp.zeros_like(diff_ref)

    diff_ref[...] += jnp.sum(dmin).reshape(1, 1)


def kernel(x, conv_w, conv_b, embed):
    x_r = x.reshape(B, C, HW)
    et2 = (-2.0 * embed).T
    kk = np.arange(K, dtype=np.float32)
    e_aug = jnp.concatenate([
        embed,
        jnp.asarray(np.floor(kk / 2.0))[None, :],
        jnp.asarray(kk % 2.0)[None, :],
        jnp.zeros((DA - D - 2, K), jnp.float32),
    ], axis=0)
    q, ind, diff = pl.pallas_call(
        _vq_body,
        grid=(B, TB),
        in_specs=[
            pl.BlockSpec((1, C, P), lambda b, t: (b, 0, t)),
            pl.BlockSpec((D, C), lambda b, t: (0, 0)),
            pl.BlockSpec((D, 1), lambda b, t: (0, 0)),
            pl.BlockSpec((K, D), lambda b, t: (0, 0)),
            pl.BlockSpec((DA, K), lambda b, t: (0, 0)),
        ],
        out_specs=[
            pl.BlockSpec((1, D, H, W), lambda b, t: (b, 0, 0, 0)),
            pl.BlockSpec((1, 1, P), lambda b, t: (b, 0, t)),
            pl.BlockSpec((1, 1), lambda b, t: (0, 0)),
        ],
        out_shape=[
            jax.ShapeDtypeStruct((B, D, H, W), jnp.float32),
            jax.ShapeDtypeStruct((B, 1, HW), jnp.int32),
            jax.ShapeDtypeStruct((1, 1), jnp.float32),
        ],
    )(x_r, conv_w, conv_b.reshape(D, 1), et2, e_aug)
    quantize = q
    embed_ind = ind.reshape(B, H, W)
    diff_s = diff[0, 0] / (B * HW * D)
    return quantize, diff_s, embed_ind
